# Initial kernel scaffold; baseline (speedup 1.0000x reference)
import jax
import jax.numpy as jnp
from jax.experimental import pallas as pl


def kernel(features, adj_spatial, adj_feature, adj_combined, enc_w1, enc_w2, dec_w1, dec_w2, wq, wk, wv, wo):
    raise NotImplementedError("write your pallas kernel here")


# speedup vs baseline: 1.4850x; 1.4850x over previous
import functools
from math import sqrt

import jax
import jax.numpy as jnp
from jax.experimental import pallas as pl
from jax.experimental.pallas import tpu as pltpu

N = 2048
IN_FEAT = 1024
HID = 512
OUT = 256
HEADS = 8
DH = OUT // HEADS
B = 256


def _dot(a, b, trans_b=False):
    dn = (((1,), (1 if trans_b else 0,)), ((), ()))
    return jax.lax.dot_general(a, b, dn, preferred_element_type=jnp.float32)


def _row_spec(cols):
    return pl.BlockSpec((B, cols), lambda i: (i, 0))


def _full_spec(rows, cols):
    return pl.BlockSpec((rows, cols), lambda i: (0, 0))


_PARAMS = pltpu.CompilerParams(dimension_semantics=("parallel",))


def _h1_body(feat_ref, w1_ref, out_ref):
    out_ref[...] = _dot(feat_ref[...], w1_ref[...])


def _enc1_body(adj_s_ref, adj_f_ref, h1_ref, w2_ref, ts_ref, tf_ref):
    h1 = h1_ref[...]
    w2 = w2_ref[...]
    ts_ref[...] = _dot(_dot(adj_s_ref[...], h1), w2)
    tf_ref[...] = _dot(_dot(adj_f_ref[...], h1), w2)


def _enc2_body(adj_s_ref, adj_f_ref, ts_ref, tf_ref, wq_ref, wk_ref, wv_ref,
               xs_ref, xf_ref, q_ref, k_ref, v_ref):
    xs = _dot(adj_s_ref[...], ts_ref[...])
    xf = _dot(adj_f_ref[...], tf_ref[...])
    xs_ref[...] = xs
    xf_ref[...] = xf
    q_ref[...] = _dot(xs, wq_ref[...])
    k_ref[...] = _dot(xf, wk_ref[...])
    v_ref[...] = _dot(xf, wv_ref[...])


def _attn_body(xs_i_ref, xf_i_ref, q_i_ref, xs_ref, xf_ref, k_ref, v_ref,
               adjc_ref, wo_ref, dw1_ref,
               srec_ref, frec_ref, latent_ref, d1_ref):
    xs_i = xs_i_ref[...]
    srec_ref[...] = jax.nn.sigmoid(_dot(xs_i, xs_ref[...], trans_b=True))
    frec_ref[...] = jax.nn.sigmoid(_dot(xf_i_ref[...], xf_ref[...], trans_b=True))

    q = q_i_ref[...]
    k = k_ref[...]
    v = v_ref[...]
    mask = adjc_ref[...] > 0.0
    scale = jnp.float32(1.0 / sqrt(DH))
    outs = []
    for h in range(HEADS):
        sl = slice(h * DH, (h + 1) * DH)
        s = _dot(q[:, sl], k[:, sl], trans_b=True) * scale
        s = jnp.where(mask, s, jnp.float32(-1e9))
        s = s - jnp.max(s, axis=-1, keepdims=True)
        e = jnp.exp(s)
        a = e / jnp.sum(e, axis=-1, keepdims=True)
        outs.append(_dot(a, v[:, sl]))
    out = jnp.concatenate(outs, axis=1)
    latent = _dot(out, wo_ref[...]) + xs_i
    latent_ref[...] = latent
    d1_ref[...] = _dot(latent, dw1_ref[...])


def _dec1_body(adj_f_ref, d1_ref, dw2_ref, r2_ref):
    r2_ref[...] = _dot(_dot(adj_f_ref[...], d1_ref[...]), dw2_ref[...])


def _dec2_body(adj_f_ref, r2_ref, recon_ref):
    recon_ref[...] = _dot(adj_f_ref[...], r2_ref[...])


def _call(body, in_specs, out_specs, out_shapes, *args):
    return pl.pallas_call(
        body,
        grid=(N // B,),
        in_specs=in_specs,
        out_specs=out_specs,
        out_shape=out_shapes,
        compiler_params=_PARAMS,
    )(*args)


def kernel(features, adj_spatial, adj_feature, adj_combined,
           enc_w1, enc_w2, dec_w1, dec_w2, wq, wk, wv, wo):
    f32 = jnp.float32

    h1 = _call(
        _h1_body,
        [_row_spec(IN_FEAT), _full_spec(IN_FEAT, HID)],
        _row_spec(HID),
        jax.ShapeDtypeStruct((N, HID), f32),
        features, enc_w1)

    ts, tf = _call(
        _enc1_body,
        [_row_spec(N), _row_spec(N), _full_spec(N, HID), _full_spec(HID, OUT)],
        [_row_spec(OUT), _row_spec(OUT)],
        [jax.ShapeDtypeStruct((N, OUT), f32)] * 2,
        adj_spatial, adj_feature, h1, enc_w2)

    xs, xf, q, k, v = _call(
        _enc2_body,
        [_row_spec(N), _row_spec(N), _full_spec(N, OUT), _full_spec(N, OUT),
         _full_spec(OUT, OUT), _full_spec(OUT, OUT), _full_spec(OUT, OUT)],
        [_row_spec(OUT)] * 5,
        [jax.ShapeDtypeStruct((N, OUT), f32)] * 5,
        adj_spatial, adj_feature, ts, tf, wq, wk, wv)

    srec, frec, latent, d1 = _call(
        _attn_body,
        [_row_spec(OUT), _row_spec(OUT), _row_spec(OUT),
         _full_spec(N, OUT), _full_spec(N, OUT), _full_spec(N, OUT),
         _full_spec(N, OUT), _row_spec(N),
         _full_spec(OUT, OUT), _full_spec(OUT, HID)],
        [_row_spec(N), _row_spec(N), _row_spec(OUT), _row_spec(HID)],
        [jax.ShapeDtypeStruct((N, N), f32), jax.ShapeDtypeStruct((N, N), f32),
         jax.ShapeDtypeStruct((N, OUT), f32), jax.ShapeDtypeStruct((N, HID), f32)],
        xs, xf, q, xs, xf, k, v, adj_combined, wo, dec_w1)

    r2 = _call(
        _dec1_body,
        [_row_spec(N), _full_spec(N, HID), _full_spec(HID, IN_FEAT)],
        _row_spec(IN_FEAT),
        jax.ShapeDtypeStruct((N, IN_FEAT), f32),
        adj_feature, d1, dec_w2)

    recon = _call(
        _dec2_body,
        [_row_spec(N), _full_spec(N, IN_FEAT)],
        _row_spec(IN_FEAT),
        jax.ShapeDtypeStruct((N, IN_FEAT), f32),
        adj_feature, r2)

    return (latent, recon, xs, xf, srec, frec)


# speedup vs baseline: 1.8625x; 1.2542x over previous
import functools
from math import sqrt

import jax
import jax.numpy as jnp
from jax.experimental import pallas as pl
from jax.experimental.pallas import tpu as pltpu

N = 2048
IN_FEAT = 1024
HID = 512
OUT = 256
HEADS = 8
DH = OUT // HEADS
B = 256


def _dot(a, b, trans_b=False):
    dn = (((1,), (1 if trans_b else 0,)), ((), ()))
    return jax.lax.dot_general(a, b, dn, preferred_element_type=jnp.float32)


def _row_spec(cols):
    return pl.BlockSpec((B, cols), lambda i: (i, 0))


def _full_spec(rows, cols):
    return pl.BlockSpec((rows, cols), lambda i: (0, 0))


_PARAMS = pltpu.CompilerParams(dimension_semantics=("parallel",))


def _h1_body(feat_ref, w1_ref, out_ref):
    out_ref[...] = _dot(feat_ref[...], w1_ref[...])


def _enc1_body(adj_s_ref, adj_f_ref, h1_ref, w2_ref, ts_ref, tf_ref):
    h1 = h1_ref[...]
    w2 = w2_ref[...]
    ts_ref[...] = _dot(_dot(adj_s_ref[...], h1), w2)
    tf_ref[...] = _dot(_dot(adj_f_ref[...], h1), w2)


def _enc2_body(adj_s_ref, adj_f_ref, ts_ref, tf_ref, wq_ref, wk_ref, wv_ref,
               xs_ref, xf_ref, q_ref, k_ref, v_ref):
    xs = _dot(adj_s_ref[...], ts_ref[...])
    xf = _dot(adj_f_ref[...], tf_ref[...])
    xs_ref[...] = xs
    xf_ref[...] = xf
    q_ref[...] = _dot(xs, wq_ref[...])
    k_ref[...] = _dot(xf, wk_ref[...])
    v_ref[...] = _dot(xf, wv_ref[...])


def _attn_body(xs_i_ref, xf_i_ref, q_i_ref, xs_ref, xf_ref, k_ref, v_ref,
               adjc_ref, wo_ref, dw1_ref,
               srec_ref, frec_ref, latent_ref, d1_ref):
    half = jnp.float32(0.5)
    xs_i = xs_i_ref[...]
    srec_ref[...] = half * jnp.tanh(half * _dot(xs_i, xs_ref[...], trans_b=True)) + half
    frec_ref[...] = half * jnp.tanh(half * _dot(xf_i_ref[...], xf_ref[...], trans_b=True)) + half

    q = q_i_ref[...] * jnp.float32(1.4426950408889634 / sqrt(DH))
    k = k_ref[...]
    v = v_ref[...]
    maskf = (adjc_ref[...] > 0.0).astype(jnp.float32)
    ones = jnp.ones((N, 1), dtype=jnp.float32)
    outs = []
    for h in range(HEADS):
        sl = slice(h * DH, (h + 1) * DH)
        s = _dot(q[:, sl], k[:, sl], trans_b=True)
        e = jnp.exp2(s - jnp.max(s, axis=-1, keepdims=True)) * maskf
        ov = _dot(e, jnp.concatenate([v[:, sl], ones], axis=1))
        outs.append(ov[:, :DH] * (1.0 / ov[:, DH:]))
    out = jnp.concatenate(outs, axis=1)
    latent = _dot(out, wo_ref[...]) + xs_i
    latent_ref[...] = latent
    d1_ref[...] = _dot(latent, dw1_ref[...])


def _dec1_body(adj_f_ref, d1_ref, dw2_ref, r2_ref):
    r2_ref[...] = _dot(_dot(adj_f_ref[...], d1_ref[...]), dw2_ref[...])


def _dec2_body(adj_f_ref, r2_ref, recon_ref):
    recon_ref[...] = _dot(adj_f_ref[...], r2_ref[...])


def _call(body, in_specs, out_specs, out_shapes, *args):
    return pl.pallas_call(
        body,
        grid=(N // B,),
        in_specs=in_specs,
        out_specs=out_specs,
        out_shape=out_shapes,
        compiler_params=_PARAMS,
    )(*args)


def kernel(features, adj_spatial, adj_feature, adj_combined,
           enc_w1, enc_w2, dec_w1, dec_w2, wq, wk, wv, wo):
    f32 = jnp.float32

    h1 = _call(
        _h1_body,
        [_row_spec(IN_FEAT), _full_spec(IN_FEAT, HID)],
        _row_spec(HID),
        jax.ShapeDtypeStruct((N, HID), f32),
        features, enc_w1)

    ts, tf = _call(
        _enc1_body,
        [_row_spec(N), _row_spec(N), _full_spec(N, HID), _full_spec(HID, OUT)],
        [_row_spec(OUT), _row_spec(OUT)],
        [jax.ShapeDtypeStruct((N, OUT), f32)] * 2,
        adj_spatial, adj_feature, h1, enc_w2)

    xs, xf, q, k, v = _call(
        _enc2_body,
        [_row_spec(N), _row_spec(N), _full_spec(N, OUT), _full_spec(N, OUT),
         _full_spec(OUT, OUT), _full_spec(OUT, OUT), _full_spec(OUT, OUT)],
        [_row_spec(OUT)] * 5,
        [jax.ShapeDtypeStruct((N, OUT), f32)] * 5,
        adj_spatial, adj_feature, ts, tf, wq, wk, wv)

    srec, frec, latent, d1 = _call(
        _attn_body,
        [_row_spec(OUT), _row_spec(OUT), _row_spec(OUT),
         _full_spec(N, OUT), _full_spec(N, OUT), _full_spec(N, OUT),
         _full_spec(N, OUT), _row_spec(N),
         _full_spec(OUT, OUT), _full_spec(OUT, HID)],
        [_row_spec(N), _row_spec(N), _row_spec(OUT), _row_spec(HID)],
        [jax.ShapeDtypeStruct((N, N), f32), jax.ShapeDtypeStruct((N, N), f32),
         jax.ShapeDtypeStruct((N, OUT), f32), jax.ShapeDtypeStruct((N, HID), f32)],
        xs, xf, q, xs, xf, k, v, adj_combined, wo, dec_w1)

    r2 = _call(
        _dec1_body,
        [_row_spec(N), _full_spec(N, HID), _full_spec(HID, IN_FEAT)],
        _row_spec(IN_FEAT),
        jax.ShapeDtypeStruct((N, IN_FEAT), f32),
        adj_feature, d1, dec_w2)

    recon = _call(
        _dec2_body,
        [_row_spec(N), _full_spec(N, IN_FEAT)],
        _row_spec(IN_FEAT),
        jax.ShapeDtypeStruct((N, IN_FEAT), f32),
        adj_feature, r2)

    return (latent, recon, xs, xf, srec, frec)
